# baseline (speedup 1.0000x reference)
import functools

import jax
import jax.numpy as jnp
from jax import lax
from jax.experimental import pallas as pl
from jax.experimental.pallas import tpu as pltpu
from jax.experimental.pallas import tpu_sc as plsc

N = 10000
HID = 128
T = 320000

NC = 2
NS = 16
NW = NC * NS
CHUNK = 128
NCHUNK = T // CHUNK
BLK = 8
CH_SPAN = 80
PADC = NW * CH_SPAN
ROWS_PER_TILE = 624
TAIL_ROWS = N - NS * ROWS_PER_TILE


def _mlps_body(x_ref, w11, b11, w12, b12, w21, b21, w22, b22, m1_ref, m2_ref):
    x = x_ref[...]
    h1 = jnp.maximum(jnp.dot(x, w11[...], preferred_element_type=jnp.float32)
                     + b11[...], 0.0)
    m1_ref[...] = (jnp.dot(h1, w12[...], preferred_element_type=jnp.float32)
                   + b12[...]).astype(jnp.bfloat16)
    h2 = jnp.maximum(jnp.dot(x, w21[...], preferred_element_type=jnp.float32)
                     + b21[...], 0.0)
    m2_ref[...] = (jnp.dot(h2, w22[...], preferred_element_type=jnp.float32)
                   + b22[...]).astype(jnp.bfloat16)


def _run_mlps(x, w11, b11, w12, b12, w21, b21, w22, b22):
    rb = 1000
    full = lambda a: pl.BlockSpec(a.shape, lambda i: (0,) * a.ndim)
    return pl.pallas_call(
        _mlps_body,
        grid=(N // rb,),
        in_specs=[pl.BlockSpec((rb, HID), lambda i: (i, 0)),
                  full(w11), full(b11), full(w12), full(b12),
                  full(w21), full(b21), full(w22), full(b22)],
        out_specs=[pl.BlockSpec((rb, HID), lambda i: (i, 0)),
                   pl.BlockSpec((rb, HID), lambda i: (i, 0))],
        out_shape=[jax.ShapeDtypeStruct((N, HID), jnp.bfloat16),
                   jax.ShapeDtypeStruct((N, HID), jnp.bfloat16)],
    )(x, w11, b11, w12, b12, w21, b21, w22, b22)


def _sc_body(m1_hbm, m2_hbm, ti0_hbm, ti1_hbm, ti2_hbm, zeros_hbm, out_hbm,
             i0_v, i1_v, i2_v, rows1_v, rows2_v, prod_v, acc_sh, gsem, ssem):
    c = lax.axis_index("c")
    s = lax.axis_index("s")
    wid = s * NC + c

    pltpu.sync_copy(zeros_hbm.at[pl.ds(0, ROWS_PER_TILE)],
                    acc_sh.at[pl.ds(s * ROWS_PER_TILE, ROWS_PER_TILE)])

    @pl.when(s == NS - 1)
    def _():
        pltpu.sync_copy(zeros_hbm.at[pl.ds(0, TAIL_ROWS)],
                        acc_sh.at[pl.ds(NS * ROWS_PER_TILE, TAIL_ROWS)])

    plsc.subcore_barrier()

    start = wid * CH_SPAN
    cnt = jnp.clip(NCHUNK - start, 0, CH_SPAN)

    def i2_row(k):
        return i2_v.at[(k // BLK) & 1, k & (BLK - 1)]

    def load_idx(k):
        blk1 = pl.ds(pl.multiple_of((start + k) * CHUNK, 8), BLK * CHUNK)
        pltpu.sync_copy(ti0_hbm.at[blk1], i0_v)
        pltpu.sync_copy(ti1_hbm.at[blk1], i1_v)
        blk2 = pl.ds(pl.multiple_of(start + k, BLK), BLK)
        pltpu.sync_copy(ti2_hbm.at[blk2], i2_v.at[(k // BLK) & 1])

    def fire_gathers(k):
        par = k & 1
        sl8 = pl.ds((k & (BLK - 1)) * CHUNK, CHUNK)
        pltpu.async_copy(m1_hbm.at[i0_v.at[sl8]], rows1_v.at[par], gsem.at[par])
        pltpu.async_copy(m2_hbm.at[i1_v.at[sl8]], rows2_v.at[par], gsem.at[par])

    def wait_gathers(k):
        par = k & 1
        sl8 = pl.ds((k & (BLK - 1)) * CHUNK, CHUNK)
        pltpu.make_async_copy(m1_hbm.at[i0_v.at[sl8]], rows1_v.at[par],
                              gsem.at[par]).wait()
        pltpu.make_async_copy(m2_hbm.at[i1_v.at[sl8]], rows2_v.at[par],
                              gsem.at[par]).wait()

    def wait_scatter(k):
        pltpu.make_async_copy(prod_v.at[k & 1], acc_sh.at[i2_row(k)],
                              ssem.at[k & 1]).wait()

    load_idx(0)
    fire_gathers(0)

    def step(k, carry):
        kk = k + 1
        in_rng = kk < cnt
        bnd8 = (kk & (BLK - 1)) == 0

        @pl.when(in_rng)
        def _():
            @pl.when(bnd8)
            def _():
                wait_gathers(k)
                load_idx(kk)

            fire_gathers(kk)

        @pl.when(jnp.logical_not(jnp.logical_and(in_rng, bnd8)))
        def _():
            wait_gathers(k)

        @pl.when(k >= 2)
        def _():
            wait_scatter(k - 2)

        par = k & 1

        @plsc.parallel_loop(0, CHUNK, 1, unroll=16)
        def mul_row(r):
            for cc in range(HID // 32):
                sl = pl.ds(cc * 32, 32)
                prod_v[par, r, sl] = rows1_v[par, r, sl] * rows2_v[par, r, sl]

        pltpu.async_copy(prod_v.at[par], acc_sh.at[i2_row(k)],
                         ssem.at[par], add=True)
        return carry

    lax.fori_loop(0, cnt, step, 0)
    wait_scatter(cnt - 2)
    wait_scatter(cnt - 1)

    plsc.subcore_barrier()
    sl = pl.ds(s * ROWS_PER_TILE, ROWS_PER_TILE)
    pltpu.sync_copy(acc_sh.at[sl], out_hbm.at[c, sl])

    @pl.when(s == NS - 1)
    def _():
        tl = pl.ds(NS * ROWS_PER_TILE, TAIL_ROWS)
        pltpu.sync_copy(acc_sh.at[tl], out_hbm.at[c, tl])


def _run_sc(m1, m2, ti0, ti1, ti2, zeros):
    mesh = plsc.VectorSubcoreMesh(core_axis_name="c", subcore_axis_name="s")
    f = pl.kernel(
        _sc_body,
        out_type=jax.ShapeDtypeStruct((NC, N, HID), jnp.bfloat16),
        mesh=mesh,
        compiler_params=pltpu.CompilerParams(use_tc_tiling_on_sc=False),
        scratch_types=[
            pltpu.VMEM((BLK * CHUNK,), jnp.int32),
            pltpu.VMEM((BLK * CHUNK,), jnp.int32),
            pltpu.VMEM((2, BLK, CHUNK), jnp.int32),
            pltpu.VMEM((2, CHUNK, HID), jnp.bfloat16),
            pltpu.VMEM((2, CHUNK, HID), jnp.bfloat16),
            pltpu.VMEM((2, CHUNK, HID), jnp.bfloat16),
            pltpu.VMEM_SHARED((N, HID), jnp.bfloat16),
            pltpu.SemaphoreType.DMA((2,)),
            pltpu.SemaphoreType.DMA((2,)),
        ],
    )
    return f(m1, m2, ti0, ti1, ti2, zeros)


def _final_body(x_ref, p_ref, nf_ref, w1x, w1m, b1, w2, b2, out_ref):
    m = (p_ref[0].astype(jnp.float32) + p_ref[1].astype(jnp.float32)) * nf_ref[...]
    h = jnp.maximum(
        jnp.dot(x_ref[...], w1x[...], preferred_element_type=jnp.float32)
        + jnp.dot(m, w1m[...], preferred_element_type=jnp.float32)
        + b1[...], 0.0)
    out_ref[...] = jnp.dot(h, w2[...], preferred_element_type=jnp.float32) + b2[...]


def _run_final(x, partials, nf, w1x, w1m, b1, w2, b2):
    rb = 1000
    full = lambda a: pl.BlockSpec(a.shape, lambda i: (0,) * a.ndim)
    return pl.pallas_call(
        _final_body,
        grid=(N // rb,),
        in_specs=[pl.BlockSpec((rb, HID), lambda i: (i, 0)),
                  pl.BlockSpec((NC, rb, HID), lambda i: (0, i, 0)),
                  pl.BlockSpec((rb, 1), lambda i: (i, 0)),
                  full(w1x), full(w1m), full(b1), full(w2), full(b2)],
        out_specs=pl.BlockSpec((rb, HID), lambda i: (i, 0)),
        out_shape=jax.ShapeDtypeStruct((N, HID), jnp.float32),
    )(x, partials, nf, w1x, w1m, b1, w2, b2)


def kernel(x, triangle_index, norm_factor,
           m1_W1, m1_b1, m1_W2, m1_b2,
           m2_W1, m2_b1, m2_W2, m2_b2,
           m3_W1, m3_b1, m3_W2, m3_b2):
    ti = triangle_index.astype(jnp.int32)
    m1, m2 = _run_mlps(x,
                       m1_W1, m1_b1.reshape(1, HID), m1_W2, m1_b2.reshape(1, HID),
                       m2_W1, m2_b1.reshape(1, HID), m2_W2, m2_b2.reshape(1, HID))
    zeros = jnp.zeros((ROWS_PER_TILE, HID), jnp.bfloat16)
    ti0 = jnp.pad(ti[0], (0, (PADC - NCHUNK) * CHUNK))
    ti1 = jnp.pad(ti[1], (0, (PADC - NCHUNK) * CHUNK))
    ti2 = jnp.pad(ti[2].reshape(NCHUNK, CHUNK), ((0, PADC - NCHUNK), (0, 0)))
    partials = _run_sc(m1, m2, ti0, ti1, ti2, zeros)
    out = _run_final(x, partials, norm_factor.reshape(N, 1),
                     m3_W1[:HID], m3_W1[HID:],
                     m3_b1.reshape(1, HID), m3_W2, m3_b2.reshape(1, HID))
    return out


# speedup vs baseline: 1.0116x; 1.0116x over previous
import functools

import jax
import jax.numpy as jnp
from jax import lax
from jax.experimental import pallas as pl
from jax.experimental.pallas import tpu as pltpu
from jax.experimental.pallas import tpu_sc as plsc

N = 10000
HID = 128
T = 320000

NC = 2
NS = 16
NW = NC * NS
CHUNK = 128
NCHUNK = T // CHUNK
BLK = 8
CH_SPAN = 80
PADC = NW * CH_SPAN
ROWS_PER_TILE = 624
TAIL_ROWS = N - NS * ROWS_PER_TILE


def _mlps_body(x_ref, w11, b11, w12, b12, w21, b21, w22, b22, m1_ref, m2_ref):
    x = x_ref[...]
    h1 = jnp.maximum(jnp.dot(x, w11[...], preferred_element_type=jnp.float32)
                     + b11[...], 0.0)
    m1_ref[...] = (jnp.dot(h1, w12[...], preferred_element_type=jnp.float32)
                   + b12[...]).astype(jnp.bfloat16)
    h2 = jnp.maximum(jnp.dot(x, w21[...], preferred_element_type=jnp.float32)
                     + b21[...], 0.0)
    m2_ref[...] = (jnp.dot(h2, w22[...], preferred_element_type=jnp.float32)
                   + b22[...]).astype(jnp.bfloat16)


def _run_mlps(x, w11, b11, w12, b12, w21, b21, w22, b22):
    rb = 1000
    full = lambda a: pl.BlockSpec(a.shape, lambda i: (0,) * a.ndim)
    return pl.pallas_call(
        _mlps_body,
        grid=(N // rb,),
        in_specs=[pl.BlockSpec((rb, HID), lambda i: (i, 0)),
                  full(w11), full(b11), full(w12), full(b12),
                  full(w21), full(b21), full(w22), full(b22)],
        out_specs=[pl.BlockSpec((rb, HID), lambda i: (i, 0)),
                   pl.BlockSpec((rb, HID), lambda i: (i, 0))],
        out_shape=[jax.ShapeDtypeStruct((N, HID), jnp.bfloat16),
                   jax.ShapeDtypeStruct((N, HID), jnp.bfloat16)],
    )(x, w11, b11, w12, b12, w21, b21, w22, b22)


def _sc_body(m1_hbm, m2_hbm, ti0_hbm, ti1_hbm, ti2_hbm, zeros_hbm, out_hbm,
             i0_v, i1_v, i2_v, rows1_v, rows2_v, prod_v, acc_sh, gsem, ssem):
    c = lax.axis_index("c")
    s = lax.axis_index("s")
    wid = s * NC + c

    pltpu.sync_copy(zeros_hbm.at[pl.ds(0, ROWS_PER_TILE)],
                    acc_sh.at[pl.ds(s * ROWS_PER_TILE, ROWS_PER_TILE)])

    @pl.when(s == NS - 1)
    def _():
        pltpu.sync_copy(zeros_hbm.at[pl.ds(0, TAIL_ROWS)],
                        acc_sh.at[pl.ds(NS * ROWS_PER_TILE, TAIL_ROWS)])

    plsc.subcore_barrier()

    start = wid * CH_SPAN
    cnt = jnp.clip(NCHUNK - start, 0, CH_SPAN)

    def i2_row(k):
        return i2_v.at[(k // BLK) & 1, k & (BLK - 1)]

    def load_idx(k):
        blk1 = pl.ds(pl.multiple_of((start + k) * CHUNK, 8), BLK * CHUNK)
        pltpu.sync_copy(ti0_hbm.at[blk1], i0_v)
        pltpu.sync_copy(ti1_hbm.at[blk1], i1_v)
        blk2 = pl.ds(pl.multiple_of(start + k, BLK), BLK)
        pltpu.sync_copy(ti2_hbm.at[blk2], i2_v.at[(k // BLK) & 1])

    def fire_gathers(k):
        par = k & 1
        sl8 = pl.ds((k & (BLK - 1)) * CHUNK, CHUNK)
        pltpu.async_copy(m1_hbm.at[i0_v.at[sl8]], rows1_v.at[par], gsem.at[par])
        pltpu.async_copy(m2_hbm.at[i1_v.at[sl8]], rows2_v.at[par], gsem.at[par])

    def wait_gathers(k):
        par = k & 1
        sl8 = pl.ds((k & (BLK - 1)) * CHUNK, CHUNK)
        pltpu.make_async_copy(m1_hbm.at[i0_v.at[sl8]], rows1_v.at[par],
                              gsem.at[par]).wait()
        pltpu.make_async_copy(m2_hbm.at[i1_v.at[sl8]], rows2_v.at[par],
                              gsem.at[par]).wait()

    def wait_scatter(k):
        pltpu.make_async_copy(prod_v.at[k & 1], acc_sh.at[i2_row(k)],
                              ssem.at[k & 1]).wait()

    load_idx(0)
    fire_gathers(0)

    def step(k, carry):
        kk = k + 1
        in_rng = kk < cnt
        bnd8 = (kk & (BLK - 1)) == 0

        @pl.when(in_rng)
        def _():
            @pl.when(bnd8)
            def _():
                wait_gathers(k)
                load_idx(kk)

            fire_gathers(kk)

        @pl.when(jnp.logical_not(jnp.logical_and(in_rng, bnd8)))
        def _():
            wait_gathers(k)

        @pl.when(k >= 2)
        def _():
            wait_scatter(k - 2)

        par = k & 1

        @plsc.parallel_loop(0, CHUNK, 1, unroll=8)
        def mul_row(r):
            for cc in range(HID // 32):
                sl = pl.ds(cc * 32, 32)
                prod_v[par, r, sl] = rows1_v[par, r, sl] * rows2_v[par, r, sl]

        pltpu.async_copy(prod_v.at[par], acc_sh.at[i2_row(k)],
                         ssem.at[par], add=True)
        return carry

    lax.fori_loop(0, cnt, step, 0)
    wait_scatter(cnt - 2)
    wait_scatter(cnt - 1)

    plsc.subcore_barrier()
    sl = pl.ds(s * ROWS_PER_TILE, ROWS_PER_TILE)
    pltpu.sync_copy(acc_sh.at[sl], out_hbm.at[c, sl])

    @pl.when(s == NS - 1)
    def _():
        tl = pl.ds(NS * ROWS_PER_TILE, TAIL_ROWS)
        pltpu.sync_copy(acc_sh.at[tl], out_hbm.at[c, tl])


def _run_sc(m1, m2, ti0, ti1, ti2, zeros):
    mesh = plsc.VectorSubcoreMesh(core_axis_name="c", subcore_axis_name="s")
    f = pl.kernel(
        _sc_body,
        out_type=jax.ShapeDtypeStruct((NC, N, HID), jnp.bfloat16),
        mesh=mesh,
        compiler_params=pltpu.CompilerParams(use_tc_tiling_on_sc=False),
        scratch_types=[
            pltpu.VMEM((BLK * CHUNK,), jnp.int32),
            pltpu.VMEM((BLK * CHUNK,), jnp.int32),
            pltpu.VMEM((2, BLK, CHUNK), jnp.int32),
            pltpu.VMEM((2, CHUNK, HID), jnp.bfloat16),
            pltpu.VMEM((2, CHUNK, HID), jnp.bfloat16),
            pltpu.VMEM((2, CHUNK, HID), jnp.bfloat16),
            pltpu.VMEM_SHARED((N, HID), jnp.bfloat16),
            pltpu.SemaphoreType.DMA((2,)),
            pltpu.SemaphoreType.DMA((2,)),
        ],
    )
    return f(m1, m2, ti0, ti1, ti2, zeros)


def _final_body(x_ref, p_ref, nf_ref, w1x, w1m, b1, w2, b2, out_ref):
    m = (p_ref[0].astype(jnp.float32) + p_ref[1].astype(jnp.float32)) * nf_ref[...]
    h = jnp.maximum(
        jnp.dot(x_ref[...], w1x[...], preferred_element_type=jnp.float32)
        + jnp.dot(m, w1m[...], preferred_element_type=jnp.float32)
        + b1[...], 0.0)
    out_ref[...] = jnp.dot(h, w2[...], preferred_element_type=jnp.float32) + b2[...]


def _run_final(x, partials, nf, w1x, w1m, b1, w2, b2):
    rb = 1000
    full = lambda a: pl.BlockSpec(a.shape, lambda i: (0,) * a.ndim)
    return pl.pallas_call(
        _final_body,
        grid=(N // rb,),
        in_specs=[pl.BlockSpec((rb, HID), lambda i: (i, 0)),
                  pl.BlockSpec((NC, rb, HID), lambda i: (0, i, 0)),
                  pl.BlockSpec((rb, 1), lambda i: (i, 0)),
                  full(w1x), full(w1m), full(b1), full(w2), full(b2)],
        out_specs=pl.BlockSpec((rb, HID), lambda i: (i, 0)),
        out_shape=jax.ShapeDtypeStruct((N, HID), jnp.float32),
    )(x, partials, nf, w1x, w1m, b1, w2, b2)


def kernel(x, triangle_index, norm_factor,
           m1_W1, m1_b1, m1_W2, m1_b2,
           m2_W1, m2_b1, m2_W2, m2_b2,
           m3_W1, m3_b1, m3_W2, m3_b2):
    ti = triangle_index.astype(jnp.int32)
    m1, m2 = _run_mlps(x,
                       m1_W1, m1_b1.reshape(1, HID), m1_W2, m1_b2.reshape(1, HID),
                       m2_W1, m2_b1.reshape(1, HID), m2_W2, m2_b2.reshape(1, HID))
    zeros = jnp.zeros((ROWS_PER_TILE, HID), jnp.bfloat16)
    ti0 = jnp.pad(ti[0], (0, (PADC - NCHUNK) * CHUNK))
    ti1 = jnp.pad(ti[1], (0, (PADC - NCHUNK) * CHUNK))
    ti2 = jnp.pad(ti[2].reshape(NCHUNK, CHUNK), ((0, PADC - NCHUNK), (0, 0)))
    partials = _run_sc(m1, m2, ti0, ti1, ti2, zeros)
    out = _run_final(x, partials, norm_factor.reshape(N, 1),
                     m3_W1[:HID], m3_W1[HID:],
                     m3_b1.reshape(1, HID), m3_W2, m3_b2.reshape(1, HID))
    return out
